# Initial kernel scaffold; baseline (speedup 1.0000x reference)
import jax
import jax.numpy as jnp
from jax.experimental import pallas as pl


def kernel(x, edge_index, positions, W_pre, b_pre, W_post, b_post, W_sc, b_sc):
    raise NotImplementedError("write your pallas kernel here")


# speedup vs baseline: 1.0015x; 1.0015x over previous
import jax
import jax.numpy as jnp
from jax.experimental import pallas as pl


def _mlp_body(x_ref, wpre_ref, bpre_ref, wpost_ref, bpost_ref, wsc_ref,
              bsc_ref, o_ref):
    x = x_ref[...]
    h = jnp.maximum(
        jnp.dot(x, wpre_ref[...], preferred_element_type=jnp.float32)
        + bpre_ref[...][None, :], 0.0)
    h = jnp.dot(h, wpost_ref[...], preferred_element_type=jnp.float32) \
        + bpost_ref[...][None, :]
    sc = jnp.dot(x, wsc_ref[...], preferred_element_type=jnp.float32) \
        + bsc_ref[...][None, :]
    o_ref[...] = sc + h


def _mlp(x, W_pre, b_pre, W_post, b_post, W_sc, b_sc):
    n, d = x.shape
    blk = 2000
    grid = (n // blk,)
    return pl.pallas_call(
        _mlp_body,
        grid=grid,
        in_specs=[
            pl.BlockSpec((blk, d), lambda i: (i, 0)),
            pl.BlockSpec((d, d), lambda i: (0, 0)),
            pl.BlockSpec((d,), lambda i: (0,)),
            pl.BlockSpec((d, d), lambda i: (0, 0)),
            pl.BlockSpec((d,), lambda i: (0,)),
            pl.BlockSpec((d, d), lambda i: (0, 0)),
            pl.BlockSpec((d,), lambda i: (0,)),
        ],
        out_specs=pl.BlockSpec((blk, d), lambda i: (i, 0)),
        out_shape=jax.ShapeDtypeStruct((n, d), jnp.float32),
    )(x, W_pre, b_pre, W_post, b_post, W_sc, b_sc)


def _spherical_harmonics(vec):
    eps = 1e-12
    n = jnp.sqrt(jnp.sum(vec * vec, axis=-1, keepdims=True))
    u = vec / jnp.maximum(n, eps)
    x = u[:, 0]; y = u[:, 1]; z = u[:, 2]
    x2 = x * x; y2 = y * y; z2 = z * z
    comps = [
        0.28209479177387814 * jnp.ones_like(x),
        0.4886025119029199 * y,
        0.4886025119029199 * z,
        0.4886025119029199 * x,
        1.0925484305920792 * x * y,
        1.0925484305920792 * y * z,
        0.31539156525252005 * (3.0 * z2 - 1.0),
        1.0925484305920792 * x * z,
        0.5462742152960396 * (x2 - y2),
        0.5900435899266435 * y * (3.0 * x2 - y2),
        2.890611442640554 * x * y * z,
        0.4570457994644658 * y * (5.0 * z2 - 1.0),
        0.3731763325901154 * z * (5.0 * z2 - 3.0),
        0.4570457994644658 * x * (5.0 * z2 - 1.0),
        1.445305721320277 * z * (x2 - y2),
        0.5900435899266435 * x * (3.0 * x2 - y2),
    ]
    return jnp.stack(comps, axis=-1)


def kernel(x, edge_index, positions, W_pre, b_pre, W_post, b_post, W_sc, b_sc):
    row = edge_index[0]
    col = edge_index[1]
    rel_pos = positions[col] - positions[row]
    sh = _spherical_harmonics(rel_pos)
    sender_features = x[row]
    tensor_prod = (sender_features[:, :, None] * sh[:, None, :]).reshape(
        sender_features.shape[0], -1)
    edge_features = jnp.concatenate([sender_features, tensor_prod], axis=1)
    aggr_out = jax.ops.segment_sum(edge_features, col, num_segments=x.shape[0])
    out = _mlp(x, W_pre, b_pre, W_post, b_post, W_sc, b_sc)
    return out, aggr_out


# speedup vs baseline: 1.5313x; 1.5290x over previous
import functools

import jax
import jax.numpy as jnp
from jax import lax
from jax.experimental import pallas as pl
from jax.experimental.pallas import tpu as pltpu
from jax.experimental.pallas import tpu_sc as plsc

N = 50000
E = 800000
D = 20
NSH = 16
ROW = D + D * NSH
NPC = 320
NCHUNK = (N + NPC - 1) // NPC
NPAD = NCHUNK * NPC
NWORK = 32
PASSES = (NCHUNK + NWORK - 1) // NWORK
SBLK = 4000
NBLK = E // SBLK
QCAP = 8192
LANECAP = 512
BATCH = 32
ACCSZ = NPC * ROW


def _sh_vectors(dx, dy, dz):
    n2 = dx * dx + dy * dy + dz * dz
    i = plsc.bitcast(n2, jnp.int32)
    i = jnp.int32(0x5F3759DF) - (i >> 1)
    r = plsc.bitcast(i, jnp.float32)
    for _ in range(3):
        r = r * (1.5 - 0.5 * n2 * r * r)
    nrm = n2 * r
    s = jnp.maximum(nrm, 1e-12)
    x = dx / s
    y = dy / s
    z = dz / s
    x2 = x * x
    y2 = y * y
    z2 = z * z
    return [
        jnp.full((16,), 0.28209479177387814, jnp.float32),
        0.4886025119029199 * y,
        0.4886025119029199 * z,
        0.4886025119029199 * x,
        1.0925484305920792 * x * y,
        1.0925484305920792 * y * z,
        0.31539156525252005 * (3.0 * z2 - 1.0),
        1.0925484305920792 * x * z,
        0.5462742152960396 * (x2 - y2),
        0.5900435899266435 * y * (3.0 * x2 - y2),
        2.890611442640554 * x * y * z,
        0.4570457994644658 * y * (5.0 * z2 - 1.0),
        0.3731763325901154 * z * (5.0 * z2 - 3.0),
        0.4570457994644658 * x * (5.0 * z2 - 1.0),
        1.445305721320277 * z * (x2 - y2),
        0.5900435899266435 * x * (3.0 * x2 - y2),
    ]


def _mesh():
    return plsc.VectorSubcoreMesh(core_axis_name="c", subcore_axis_name="s")


def _scan_body(row_hbm, col_hbm, qpk_hbm, cnt_hbm, colbuf, rowbuf,
               qr, qc, scb, cbuf, csem, rsem):
    cid = lax.axis_index("c")
    sid = lax.axis_index("s")
    wid = sid * 2 + cid
    iota = lax.broadcasted_iota(jnp.int32, (16,), 0)
    zvec = jnp.zeros((16,), jnp.int32)

    def issue_scan_st(b, off):
        pltpu.async_copy(col_hbm.at[pl.ds(b * SBLK, SBLK)],
                         colbuf.at[pl.ds(off * SBLK, SBLK)], csem)
        pltpu.async_copy(row_hbm.at[pl.ds(b * SBLK, SBLK)],
                         rowbuf.at[pl.ds(off * SBLK, SBLK)], rsem)

    def issue_scan(b, par):
        @pl.when(par == 0)
        def _():
            issue_scan_st(b, 0)

        @pl.when(par != 0)
        def _():
            issue_scan_st(b, 1)

    def wait_scan_st(off):
        pltpu.make_async_copy(col_hbm.at[pl.ds(0, SBLK)],
                              colbuf.at[pl.ds(off * SBLK, SBLK)], csem).wait()
        pltpu.make_async_copy(row_hbm.at[pl.ds(0, SBLK)],
                              rowbuf.at[pl.ds(off * SBLK, SBLK)], rsem).wait()

    def wait_scan(par):
        @pl.when(par == 0)
        def _():
            wait_scan_st(0)

        @pl.when(par != 0)
        def _():
            wait_scan_st(1)

    def qz(zi, _):
        qr[pl.ds(zi * 16, 16)] = zvec
        qc[pl.ds(zi * 16, 16)] = zvec
        return 0
    lax.fori_loop(0, QCAP // 16, qz, 0)
    dz = qr[pl.ds(0, 16)][0]

    def pass_body(p, _):
        chunk = p * NWORK + wid
        lo = chunk * NPC

        @pl.when(chunk < NCHUNK)
        def _run():
            issue_scan_st(0, 0)

            def blk(b, cnt):
                par = lax.rem(b, 2)

                @pl.when(b + 1 < NBLK)
                def _():
                    issue_scan(b + 1, lax.rem(b + 1, 2))
                wait_scan(par)

                pbase = par * SBLK

                def vec(v, cnt):
                    cv = colbuf[pl.ds(pbase + v * 16, 16)]
                    rv = rowbuf[pl.ds(pbase + v * 16, 16)]
                    hit = (cv >= lo) & (cv < lo + NPC)
                    packed = ((cv - lo) << 16) | rv
                    offs = jnp.where(hit, cnt * 16 + iota,
                                     QCAP - 16 + iota)
                    plsc.store_scatter(qr, [offs], packed)
                    return jnp.minimum(cnt + hit.astype(jnp.int32),
                                       LANECAP - 1)
                return lax.fori_loop(0, SBLK // 16 + dz, vec, cnt)

            cnt = lax.fori_loop(0, NBLK + dz, blk, zvec)
            pltpu.sync_copy(qr, qpk_hbm.at[pl.ds(chunk * QCAP, QCAP)])
            cbuf[pl.ds(0, 16)] = cnt
            pltpu.sync_copy(cbuf, cnt_hbm.at[pl.ds(chunk * 16, 16)])
        return 0
    lax.fori_loop(0, PASSES + dz, pass_body, 0)


def _aggr_body(tbl_hbm, qpk_hbm, cnt_hbm, aggr_hbm, acc,
               qp, idxr, idxc, ts, tc, cbuf, qsem, xsem, pcsem):
    cid = lax.axis_index("c")
    sid = lax.axis_index("s")
    wid = sid * 2 + cid
    iota = lax.broadcasted_iota(jnp.int32, (16,), 0)
    zvec = jnp.zeros((16,), jnp.int32)
    acc[pl.ds(0, 16)] = jnp.zeros((16,), jnp.float32)
    dz = plsc.bitcast(acc[pl.ds(0, 16)], jnp.int32)[0]

    def pass_body(p, _):
        chunk = p * NWORK + wid
        lo = chunk * NPC

        @pl.when(chunk < NCHUNK)
        def _run():
            pltpu.sync_copy(cnt_hbm.at[pl.ds(chunk * 16, 16)], cbuf)
            cntv = cbuf[pl.ds(0, 16)]
            h1 = pltpu.async_copy(qpk_hbm.at[pl.ds(chunk * QCAP, QCAP)],
                                  qp, qsem)

            def zbody(zi, _):
                acc[pl.ds(zi * 16, 16)] = jnp.zeros((16,), jnp.float32)
                return 0
            lax.fori_loop(0, (ACCSZ + 352) // 16 + dz, zbody, 0)
            h1.wait()

            nbs = cntv[0]
            for k in range(1, 16):
                nbs = jnp.maximum(nbs, cntv[k])
            nb = (nbs + (BATCH // 16) - 1) // (BATCH // 16)

            def batch_body(b, _):
                qoff = b * BATCH

                def unpack(k, _):
                    pv = qp[pl.ds(qoff + k * 16, 16)]
                    idxr[pl.ds(k * 16, 16)] = pv & jnp.int32(0xFFFF)
                    idxc[pl.ds(k * 16, 16)] = (pv >> 16) + lo
                    return 0
                lax.fori_loop(0, BATCH // 16 + dz, unpack, 0)
                hx = pltpu.async_copy(
                    tbl_hbm.at[idxr.at[pl.ds(0, BATCH)]], ts, xsem)
                hc = pltpu.async_copy(
                    tbl_hbm.at[idxc.at[pl.ds(0, BATCH)]], tc, pcsem)
                hx.wait()
                hc.wait()

                def group_body(g, _):
                    b16 = g * 16
                    qo = qoff + b16
                    slot = b * (BATCH // 16) + g
                    msk = cntv > slot
                    pv = qp[pl.ds(qo, 16)]
                    addr = jnp.where(msk, (pv >> 16) * ROW,
                                     jnp.int32(ACCSZ))
                    gi = b16 + iota
                    dx = (plsc.load_gather(tc, [gi, zvec + D])
                          - plsc.load_gather(ts, [gi, zvec + D]))
                    dy = (plsc.load_gather(tc, [gi, zvec + (D + 1)])
                          - plsc.load_gather(ts, [gi, zvec + (D + 1)]))
                    dz = (plsc.load_gather(tc, [gi, zvec + (D + 2)])
                          - plsc.load_gather(ts, [gi, zvec + (D + 2)]))
                    sh = _sh_vectors(dx, dy, dz)

                    def i_body(i, _):
                        sv = plsc.load_gather(ts, [gi, zvec + i])
                        plsc.addupdate_scatter(acc, [addr + i], sv)
                        a2 = addr + (D + 16 * i)
                        for j in range(16):
                            plsc.addupdate_scatter(acc, [a2 + j], sv * sh[j])
                        return 0
                    lax.fori_loop(0, D, i_body, 0)
                    return 0
                lax.fori_loop(0, BATCH // 16 + dz, group_body, 0)
                return 0
            lax.fori_loop(0, nb, batch_body, 0)
            pltpu.sync_copy(acc.at[pl.ds(0, ACCSZ)],
                            aggr_hbm.at[pl.ds(chunk * ACCSZ, ACCSZ)])
        return 0
    lax.fori_loop(0, PASSES + dz, pass_body, 0)


def _sc_scan(row, col):
    f = functools.partial(
        pl.kernel,
        mesh=_mesh(),
        compiler_params=pltpu.CompilerParams(needs_layout_passes=False),
        out_type=(
            jax.ShapeDtypeStruct((NCHUNK * QCAP,), jnp.int32),
            jax.ShapeDtypeStruct((NCHUNK * 16,), jnp.int32),
        ),
        scratch_types=[
            pltpu.VMEM((2 * SBLK,), jnp.int32),
            pltpu.VMEM((2 * SBLK,), jnp.int32),
            pltpu.VMEM((QCAP,), jnp.int32),
            pltpu.VMEM((QCAP,), jnp.int32),
            pltpu.VMEM((16,), jnp.int32),
            pltpu.VMEM((16,), jnp.int32),
            pltpu.SemaphoreType.DMA,
            pltpu.SemaphoreType.DMA,
        ],
    )(_scan_body)
    return f(row, col)


def _sc_aggregate(tbl, qpk, cnt):
    f = functools.partial(
        pl.kernel,
        mesh=_mesh(),
        compiler_params=pltpu.CompilerParams(needs_layout_passes=False),
        out_type=jax.ShapeDtypeStruct((NPAD * ROW,), jnp.float32),
        scratch_types=[
            pltpu.VMEM((ACCSZ + 352,), jnp.float32),
            pltpu.VMEM((QCAP,), jnp.int32),
            pltpu.VMEM((BATCH,), jnp.int32),
            pltpu.VMEM((BATCH,), jnp.int32),
            pltpu.VMEM((BATCH, 128), jnp.float32),
            pltpu.VMEM((BATCH, 128), jnp.float32),
            pltpu.VMEM((16,), jnp.int32),
            pltpu.SemaphoreType.DMA,
            pltpu.SemaphoreType.DMA,
            pltpu.SemaphoreType.DMA,
        ],
    )(_aggr_body)
    return f(tbl, qpk, cnt)


def _mlp_body(x_ref, wpre_ref, bpre_ref, wpost_ref, bpost_ref, wsc_ref,
              bsc_ref, o_ref):
    x = x_ref[...]
    h = jnp.maximum(
        jnp.dot(x, wpre_ref[...], preferred_element_type=jnp.float32)
        + bpre_ref[...][None, :], 0.0)
    h = jnp.dot(h, wpost_ref[...], preferred_element_type=jnp.float32) \
        + bpost_ref[...][None, :]
    sc = jnp.dot(x, wsc_ref[...], preferred_element_type=jnp.float32) \
        + bsc_ref[...][None, :]
    o_ref[...] = sc + h


def _mlp(x, W_pre, b_pre, W_post, b_post, W_sc, b_sc):
    n, d = x.shape
    blk = 2000
    return pl.pallas_call(
        _mlp_body,
        grid=(n // blk,),
        in_specs=[
            pl.BlockSpec((blk, d), lambda i: (i, 0)),
            pl.BlockSpec((d, d), lambda i: (0, 0)),
            pl.BlockSpec((d,), lambda i: (0,)),
            pl.BlockSpec((d, d), lambda i: (0, 0)),
            pl.BlockSpec((d,), lambda i: (0,)),
            pl.BlockSpec((d, d), lambda i: (0, 0)),
            pl.BlockSpec((d,), lambda i: (0,)),
        ],
        out_specs=pl.BlockSpec((blk, d), lambda i: (i, 0)),
        out_shape=jax.ShapeDtypeStruct((n, d), jnp.float32),
    )(x, W_pre, b_pre, W_post, b_post, W_sc, b_sc)


def kernel(x, edge_index, positions, W_pre, b_pre, W_post, b_post, W_sc,
           b_sc):
    row = edge_index[0]
    col = edge_index[1]
    qpk, cnt = _sc_scan(row, col)
    tbl = jnp.pad(jnp.concatenate([x, positions], axis=1),
                  ((0, 0), (0, 128 - D - 3)))
    aggr = _sc_aggregate(tbl, qpk, cnt)
    aggr = aggr.reshape(NPAD, ROW)[:N]
    out = _mlp(x, W_pre, b_pre, W_post, b_post, W_sc, b_sc)
    return out, aggr


# speedup vs baseline: 1.7102x; 1.1168x over previous
import functools

import jax
import jax.numpy as jnp
from jax import lax
from jax.experimental import pallas as pl
from jax.experimental.pallas import tpu as pltpu
from jax.experimental.pallas import tpu_sc as plsc

N = 50000
E = 800000
D = 20
NSH = 16
ROW = D + D * NSH
NPC = 320
NCHUNK = (N + NPC - 1) // NPC
NPAD = NCHUNK * NPC
NWORK = 32
PASSES = (NCHUNK + NWORK - 1) // NWORK
SBLK = 4000
NBLK = E // SBLK
QCAP = 8192
LANECAP = 512
BATCH = 16
ACCSZ = NPC * ROW


def _sh_vectors(dx, dy, dz):
    n2 = dx * dx + dy * dy + dz * dz
    i = plsc.bitcast(n2, jnp.int32)
    i = jnp.int32(0x5F3759DF) - (i >> 1)
    r = plsc.bitcast(i, jnp.float32)
    for _ in range(3):
        r = r * (1.5 - 0.5 * n2 * r * r)
    nrm = n2 * r
    s = jnp.maximum(nrm, 1e-12)
    x = dx / s
    y = dy / s
    z = dz / s
    x2 = x * x
    y2 = y * y
    z2 = z * z
    return [
        jnp.full((16,), 0.28209479177387814, jnp.float32),
        0.4886025119029199 * y,
        0.4886025119029199 * z,
        0.4886025119029199 * x,
        1.0925484305920792 * x * y,
        1.0925484305920792 * y * z,
        0.31539156525252005 * (3.0 * z2 - 1.0),
        1.0925484305920792 * x * z,
        0.5462742152960396 * (x2 - y2),
        0.5900435899266435 * y * (3.0 * x2 - y2),
        2.890611442640554 * x * y * z,
        0.4570457994644658 * y * (5.0 * z2 - 1.0),
        0.3731763325901154 * z * (5.0 * z2 - 3.0),
        0.4570457994644658 * x * (5.0 * z2 - 1.0),
        1.445305721320277 * z * (x2 - y2),
        0.5900435899266435 * x * (3.0 * x2 - y2),
    ]


def _mesh():
    return plsc.VectorSubcoreMesh(core_axis_name="c", subcore_axis_name="s")


def _scan_body(row_hbm, col_hbm, qpk_hbm, cnt_hbm, colbuf, rowbuf,
               qr, qc, scb, cbuf, csem, rsem):
    cid = lax.axis_index("c")
    sid = lax.axis_index("s")
    wid = sid * 2 + cid
    iota = lax.broadcasted_iota(jnp.int32, (16,), 0)
    zvec = jnp.zeros((16,), jnp.int32)

    def issue_scan_st(b, off):
        pltpu.async_copy(col_hbm.at[pl.ds(b * SBLK, SBLK)],
                         colbuf.at[pl.ds(off * SBLK, SBLK)], csem)
        pltpu.async_copy(row_hbm.at[pl.ds(b * SBLK, SBLK)],
                         rowbuf.at[pl.ds(off * SBLK, SBLK)], rsem)

    def issue_scan(b, par):
        @pl.when(par == 0)
        def _():
            issue_scan_st(b, 0)

        @pl.when(par != 0)
        def _():
            issue_scan_st(b, 1)

    def wait_scan_st(off):
        pltpu.make_async_copy(col_hbm.at[pl.ds(0, SBLK)],
                              colbuf.at[pl.ds(off * SBLK, SBLK)], csem).wait()
        pltpu.make_async_copy(row_hbm.at[pl.ds(0, SBLK)],
                              rowbuf.at[pl.ds(off * SBLK, SBLK)], rsem).wait()

    def wait_scan(par):
        @pl.when(par == 0)
        def _():
            wait_scan_st(0)

        @pl.when(par != 0)
        def _():
            wait_scan_st(1)

    def qz(zi, _):
        qr[pl.ds(zi * 16, 16)] = zvec
        qc[pl.ds(zi * 16, 16)] = zvec
        return 0
    lax.fori_loop(0, QCAP // 16, qz, 0)
    dz = qr[pl.ds(0, 16)][0]

    def pass_body(p, _):
        chunk = p * NWORK + wid
        lo = chunk * NPC

        @pl.when(chunk < NCHUNK)
        def _run():
            issue_scan_st(0, 0)

            def blk(b, cnt):
                par = lax.rem(b, 2)

                @pl.when(b + 1 < NBLK)
                def _():
                    issue_scan(b + 1, lax.rem(b + 1, 2))
                wait_scan(par)

                pbase = par * SBLK

                def vec(v, cnt):
                    cv = colbuf[pl.ds(pbase + v * 16, 16)]
                    rv = rowbuf[pl.ds(pbase + v * 16, 16)]
                    hit = (cv >= lo) & (cv < lo + NPC)
                    packed = ((cv - lo) << 16) | rv
                    offs = jnp.where(hit, cnt * 16 + iota,
                                     QCAP - 16 + iota)
                    plsc.store_scatter(qr, [offs], packed)
                    return jnp.minimum(cnt + hit.astype(jnp.int32),
                                       LANECAP - 1)
                return lax.fori_loop(0, SBLK // 16 + dz, vec, cnt)

            cnt = lax.fori_loop(0, NBLK + dz, blk, zvec)
            pltpu.sync_copy(qr, qpk_hbm.at[pl.ds(chunk * QCAP, QCAP)])
            cbuf[pl.ds(0, 16)] = cnt
            pltpu.sync_copy(cbuf, cnt_hbm.at[pl.ds(chunk * 16, 16)])
        return 0
    lax.fori_loop(0, PASSES + dz, pass_body, 0)


def _aggr_body(tbl_hbm, qpk_hbm, cnt_hbm, aggr_hbm, acc,
               qp, idxr, idxc, ts, tc, cbuf, qsem, xsem, pcsem):
    cid = lax.axis_index("c")
    sid = lax.axis_index("s")
    wid = sid * 2 + cid
    iota = lax.broadcasted_iota(jnp.int32, (16,), 0)
    zvec = jnp.zeros((16,), jnp.int32)
    acc[pl.ds(0, 16)] = jnp.zeros((16,), jnp.float32)
    dz2 = plsc.bitcast(acc[pl.ds(0, 16)], jnp.int32)[0]

    def pass_body(p, _):
        chunk = p * NWORK + wid
        lo = chunk * NPC

        @pl.when(chunk < NCHUNK)
        def _run():
            pltpu.sync_copy(cnt_hbm.at[pl.ds(chunk * 16, 16)], cbuf)
            cntv = cbuf[pl.ds(0, 16)]
            h1 = pltpu.async_copy(qpk_hbm.at[pl.ds(chunk * QCAP, QCAP)],
                                  qp, qsem)

            def zbody(zi, _):
                acc[pl.ds(zi * 16, 16)] = jnp.zeros((16,), jnp.float32)
                return 0
            lax.fori_loop(0, (ACCSZ + 352) // 16 + dz2, zbody, 0)
            h1.wait()

            nbs = cntv[0]
            for k in range(1, 16):
                nbs = jnp.maximum(nbs, cntv[k])

            def issue_slot(b, off):
                pv = qp[pl.ds(b * 16, 16)]
                idxr[pl.ds(off * 16, 16)] = pv & jnp.int32(0xFFFF)
                idxc[pl.ds(off * 16, 16)] = (pv >> 16) + lo
                pltpu.async_copy(
                    tbl_hbm.at[idxr.at[pl.ds(off * 16, 16)]],
                    ts.at[pl.ds(off * 16, 16)], xsem)
                pltpu.async_copy(
                    tbl_hbm.at[idxc.at[pl.ds(off * 16, 16)]],
                    tc.at[pl.ds(off * 16, 16)], pcsem)

            def issue_dyn(b, par):
                @pl.when(par == 0)
                def _():
                    issue_slot(b, 0)

                @pl.when(par != 0)
                def _():
                    issue_slot(b, 1)

            def wait_slot(off):
                pltpu.make_async_copy(
                    tbl_hbm.at[idxr.at[pl.ds(off * 16, 16)]],
                    ts.at[pl.ds(off * 16, 16)], xsem).wait()
                pltpu.make_async_copy(
                    tbl_hbm.at[idxc.at[pl.ds(off * 16, 16)]],
                    tc.at[pl.ds(off * 16, 16)], pcsem).wait()

            def wait_dyn(par):
                @pl.when(par == 0)
                def _():
                    wait_slot(0)

                @pl.when(par != 0)
                def _():
                    wait_slot(1)

            @pl.when(nbs > 0)
            def _():
                issue_slot(0, 0)

            def sbody(b, _):
                par = lax.rem(b, 2)

                @pl.when(b + 1 < nbs)
                def _():
                    issue_dyn(b + 1, lax.rem(b + 1, 2))
                wait_dyn(par)
                msk = cntv > b
                pv = qp[pl.ds(b * 16, 16)]
                addr = jnp.where(msk, (pv >> 16) * ROW, jnp.int32(ACCSZ))
                gi = par * 16 + iota
                dx = (plsc.load_gather(tc, [gi, zvec + D])
                      - plsc.load_gather(ts, [gi, zvec + D]))
                dy = (plsc.load_gather(tc, [gi, zvec + (D + 1)])
                      - plsc.load_gather(ts, [gi, zvec + (D + 1)]))
                dz = (plsc.load_gather(tc, [gi, zvec + (D + 2)])
                      - plsc.load_gather(ts, [gi, zvec + (D + 2)]))
                sh = _sh_vectors(dx, dy, dz)

                def i_body(i, _):
                    sv = plsc.load_gather(ts, [gi, zvec + i])
                    plsc.addupdate_scatter(acc, [addr + i], sv)
                    a2 = addr + (D + 16 * i)
                    for j in range(16):
                        plsc.addupdate_scatter(acc, [a2 + j], sv * sh[j])
                    return 0
                lax.fori_loop(0, D + dz2, i_body, 0)
                return 0
            lax.fori_loop(0, nbs, sbody, 0)
            pltpu.sync_copy(acc.at[pl.ds(0, ACCSZ)],
                            aggr_hbm.at[pl.ds(chunk * ACCSZ, ACCSZ)])
        return 0
    lax.fori_loop(0, PASSES + dz2, pass_body, 0)


def _sc_scan(row, col):
    f = functools.partial(
        pl.kernel,
        mesh=_mesh(),
        compiler_params=pltpu.CompilerParams(needs_layout_passes=False),
        out_type=(
            jax.ShapeDtypeStruct((NCHUNK * QCAP,), jnp.int32),
            jax.ShapeDtypeStruct((NCHUNK * 16,), jnp.int32),
        ),
        scratch_types=[
            pltpu.VMEM((2 * SBLK,), jnp.int32),
            pltpu.VMEM((2 * SBLK,), jnp.int32),
            pltpu.VMEM((QCAP,), jnp.int32),
            pltpu.VMEM((QCAP,), jnp.int32),
            pltpu.VMEM((16,), jnp.int32),
            pltpu.VMEM((16,), jnp.int32),
            pltpu.SemaphoreType.DMA,
            pltpu.SemaphoreType.DMA,
        ],
    )(_scan_body)
    return f(row, col)


def _sc_aggregate(tbl, qpk, cnt):
    f = functools.partial(
        pl.kernel,
        mesh=_mesh(),
        compiler_params=pltpu.CompilerParams(needs_layout_passes=False),
        out_type=jax.ShapeDtypeStruct((NPAD * ROW,), jnp.float32),
        scratch_types=[
            pltpu.VMEM((ACCSZ + 352,), jnp.float32),
            pltpu.VMEM((QCAP,), jnp.int32),
            pltpu.VMEM((2 * BATCH,), jnp.int32),
            pltpu.VMEM((2 * BATCH,), jnp.int32),
            pltpu.VMEM((2 * BATCH, 128), jnp.float32),
            pltpu.VMEM((2 * BATCH, 128), jnp.float32),
            pltpu.VMEM((16,), jnp.int32),
            pltpu.SemaphoreType.DMA,
            pltpu.SemaphoreType.DMA,
            pltpu.SemaphoreType.DMA,
        ],
    )(_aggr_body)
    return f(tbl, qpk, cnt)


def _mlp_body(x_ref, wpre_ref, bpre_ref, wpost_ref, bpost_ref, wsc_ref,
              bsc_ref, o_ref):
    x = x_ref[...]
    h = jnp.maximum(
        jnp.dot(x, wpre_ref[...], preferred_element_type=jnp.float32)
        + bpre_ref[...][None, :], 0.0)
    h = jnp.dot(h, wpost_ref[...], preferred_element_type=jnp.float32) \
        + bpost_ref[...][None, :]
    sc = jnp.dot(x, wsc_ref[...], preferred_element_type=jnp.float32) \
        + bsc_ref[...][None, :]
    o_ref[...] = sc + h


def _mlp(x, W_pre, b_pre, W_post, b_post, W_sc, b_sc):
    n, d = x.shape
    blk = 2000
    return pl.pallas_call(
        _mlp_body,
        grid=(n // blk,),
        in_specs=[
            pl.BlockSpec((blk, d), lambda i: (i, 0)),
            pl.BlockSpec((d, d), lambda i: (0, 0)),
            pl.BlockSpec((d,), lambda i: (0,)),
            pl.BlockSpec((d, d), lambda i: (0, 0)),
            pl.BlockSpec((d,), lambda i: (0,)),
            pl.BlockSpec((d, d), lambda i: (0, 0)),
            pl.BlockSpec((d,), lambda i: (0,)),
        ],
        out_specs=pl.BlockSpec((blk, d), lambda i: (i, 0)),
        out_shape=jax.ShapeDtypeStruct((n, d), jnp.float32),
    )(x, W_pre, b_pre, W_post, b_post, W_sc, b_sc)


def kernel(x, edge_index, positions, W_pre, b_pre, W_post, b_post, W_sc,
           b_sc):
    row = edge_index[0]
    col = edge_index[1]
    qpk, cnt = _sc_scan(row, col)
    tbl = jnp.pad(jnp.concatenate([x, positions], axis=1),
                  ((0, 0), (0, 128 - D - 3)))
    aggr = _sc_aggregate(tbl, qpk, cnt)
    aggr = aggr.reshape(NPAD, ROW)[:N]
    out = _mlp(x, W_pre, b_pre, W_post, b_post, W_sc, b_sc)
    return out, aggr


# speedup vs baseline: 1.7446x; 1.0201x over previous
import functools

import jax
import jax.numpy as jnp
from jax import lax
from jax.experimental import pallas as pl
from jax.experimental.pallas import tpu as pltpu
from jax.experimental.pallas import tpu_sc as plsc

N = 50000
E = 800000
D = 20
NSH = 16
ROW = D + D * NSH
NPC = 320
NCHUNK = (N + NPC - 1) // NPC
NPAD = NCHUNK * NPC
NWORK = 32
PASSES = (NCHUNK + NWORK - 1) // NWORK
SBLK = 4000
NBLK = E // SBLK
QCAP = 8192
LANECAP = 512
BATCH = 16
ACCSZ = NPC * ROW


def _sh_vectors(dx, dy, dz):
    n2 = dx * dx + dy * dy + dz * dz
    i = plsc.bitcast(n2, jnp.int32)
    i = jnp.int32(0x5F3759DF) - (i >> 1)
    r = plsc.bitcast(i, jnp.float32)
    for _ in range(3):
        r = r * (1.5 - 0.5 * n2 * r * r)
    nrm = n2 * r
    s = jnp.maximum(nrm, 1e-12)
    x = dx / s
    y = dy / s
    z = dz / s
    x2 = x * x
    y2 = y * y
    z2 = z * z
    return [
        jnp.full((16,), 0.28209479177387814, jnp.float32),
        0.4886025119029199 * y,
        0.4886025119029199 * z,
        0.4886025119029199 * x,
        1.0925484305920792 * x * y,
        1.0925484305920792 * y * z,
        0.31539156525252005 * (3.0 * z2 - 1.0),
        1.0925484305920792 * x * z,
        0.5462742152960396 * (x2 - y2),
        0.5900435899266435 * y * (3.0 * x2 - y2),
        2.890611442640554 * x * y * z,
        0.4570457994644658 * y * (5.0 * z2 - 1.0),
        0.3731763325901154 * z * (5.0 * z2 - 3.0),
        0.4570457994644658 * x * (5.0 * z2 - 1.0),
        1.445305721320277 * z * (x2 - y2),
        0.5900435899266435 * x * (3.0 * x2 - y2),
    ]


def _mesh():
    return plsc.VectorSubcoreMesh(core_axis_name="c", subcore_axis_name="s")


def _scan_body(row_hbm, col_hbm, qpk_hbm, cnt_hbm, colbuf, rowbuf,
               qr, qc, scb, cbuf, csem, rsem):
    cid = lax.axis_index("c")
    sid = lax.axis_index("s")
    wid = sid * 2 + cid
    iota = lax.broadcasted_iota(jnp.int32, (16,), 0)
    zvec = jnp.zeros((16,), jnp.int32)

    def issue_scan_st(b, off):
        pltpu.async_copy(col_hbm.at[pl.ds(b * SBLK, SBLK)],
                         colbuf.at[pl.ds(off * SBLK, SBLK)], csem)
        pltpu.async_copy(row_hbm.at[pl.ds(b * SBLK, SBLK)],
                         rowbuf.at[pl.ds(off * SBLK, SBLK)], rsem)

    def issue_scan(b, par):
        @pl.when(par == 0)
        def _():
            issue_scan_st(b, 0)

        @pl.when(par != 0)
        def _():
            issue_scan_st(b, 1)

    def wait_scan_st(off):
        pltpu.make_async_copy(col_hbm.at[pl.ds(0, SBLK)],
                              colbuf.at[pl.ds(off * SBLK, SBLK)], csem).wait()
        pltpu.make_async_copy(row_hbm.at[pl.ds(0, SBLK)],
                              rowbuf.at[pl.ds(off * SBLK, SBLK)], rsem).wait()

    def wait_scan(par):
        @pl.when(par == 0)
        def _():
            wait_scan_st(0)

        @pl.when(par != 0)
        def _():
            wait_scan_st(1)

    def qz(zi, _):
        qr[pl.ds(zi * 16, 16)] = zvec
        qc[pl.ds(zi * 16, 16)] = zvec
        return 0
    lax.fori_loop(0, QCAP // 16, qz, 0)
    dz = qr[pl.ds(0, 16)][0]

    def pass_body(p, _):
        chunk = p * NWORK + wid
        lo = chunk * NPC

        @pl.when(chunk < NCHUNK)
        def _run():
            issue_scan_st(0, 0)

            def blk(b, cnt):
                par = lax.rem(b, 2)

                @pl.when(b + 1 < NBLK)
                def _():
                    issue_scan(b + 1, lax.rem(b + 1, 2))
                wait_scan(par)

                pbase = par * SBLK

                def vec(v, cnt):
                    cv = colbuf[pl.ds(pbase + v * 16, 16)]
                    rv = rowbuf[pl.ds(pbase + v * 16, 16)]
                    hit = (cv >= lo) & (cv < lo + NPC)
                    packed = ((cv - lo) << 16) | rv
                    offs = jnp.where(hit, cnt * 16 + iota,
                                     QCAP - 16 + iota)
                    plsc.store_scatter(qr, [offs], packed)
                    return jnp.minimum(cnt + hit.astype(jnp.int32),
                                       LANECAP - 1)
                return lax.fori_loop(0, SBLK // 16 + dz, vec, cnt)

            cnt = lax.fori_loop(0, NBLK + dz, blk, zvec)
            pltpu.sync_copy(qr, qpk_hbm.at[pl.ds(chunk * QCAP, QCAP)])
            cbuf[pl.ds(0, 16)] = cnt
            pltpu.sync_copy(cbuf, cnt_hbm.at[pl.ds(chunk * 16, 16)])
        return 0
    lax.fori_loop(0, PASSES + dz, pass_body, 0)


def _aggr_body(tbl_hbm, pos_hbm, qpk_hbm, cnt_hbm, aggr_hbm, acc,
               qp, idxr, ts, poscb, cbuf, qsem, xsem, pcsem):
    cid = lax.axis_index("c")
    sid = lax.axis_index("s")
    wid = sid * 2 + cid
    iota = lax.broadcasted_iota(jnp.int32, (16,), 0)
    zvec = jnp.zeros((16,), jnp.int32)
    acc[pl.ds(0, 16)] = jnp.zeros((16,), jnp.float32)
    dz2 = plsc.bitcast(acc[pl.ds(0, 16)], jnp.int32)[0]

    def pass_body(p, _):
        chunk = p * NWORK + wid
        lo = chunk * NPC

        @pl.when(chunk < NCHUNK)
        def _run():
            pltpu.sync_copy(cnt_hbm.at[pl.ds(chunk * 16, 16)], cbuf)
            pltpu.sync_copy(pos_hbm.at[pl.ds(lo * 8, NPC * 8)], poscb)
            cntv = cbuf[pl.ds(0, 16)]
            h1 = pltpu.async_copy(qpk_hbm.at[pl.ds(chunk * QCAP, QCAP)],
                                  qp, qsem)

            def zbody(zi, _):
                acc[pl.ds(zi * 16, 16)] = jnp.zeros((16,), jnp.float32)
                return 0
            lax.fori_loop(0, (ACCSZ + 352) // 16 + dz2, zbody, 0)
            h1.wait()

            nbs = cntv[0]
            for k in range(1, 16):
                nbs = jnp.maximum(nbs, cntv[k])

            def issue_slot(b, off):
                pv = qp[pl.ds(b * 16, 16)]
                idxr[pl.ds(off * 16, 16)] = pv & jnp.int32(0xFFFF)
                pltpu.async_copy(
                    tbl_hbm.at[idxr.at[pl.ds(off * 16, 16)]],
                    ts.at[pl.ds(off * 16, 16)], xsem)

            def issue_dyn(b, par):
                @pl.when(par == 0)
                def _():
                    issue_slot(b, 0)

                @pl.when(par != 0)
                def _():
                    issue_slot(b, 1)

            def wait_slot(off):
                pltpu.make_async_copy(
                    tbl_hbm.at[idxr.at[pl.ds(off * 16, 16)]],
                    ts.at[pl.ds(off * 16, 16)], xsem).wait()

            def wait_dyn(par):
                @pl.when(par == 0)
                def _():
                    wait_slot(0)

                @pl.when(par != 0)
                def _():
                    wait_slot(1)

            @pl.when(nbs > 0)
            def _():
                issue_slot(0, 0)

            def sbody(b, _):
                par = lax.rem(b, 2)

                @pl.when(b + 1 < nbs)
                def _():
                    issue_dyn(b + 1, lax.rem(b + 1, 2))
                wait_dyn(par)
                msk = cntv > b
                pv = qp[pl.ds(b * 16, 16)]
                cl = pv >> 16
                addr = jnp.where(msk, cl * ROW, jnp.int32(ACCSZ))
                gi = par * 16 + iota
                cl8 = cl * 8
                dx = (plsc.load_gather(poscb, [cl8])
                      - plsc.load_gather(ts, [gi, zvec + D]))
                dy = (plsc.load_gather(poscb, [cl8 + 1])
                      - plsc.load_gather(ts, [gi, zvec + (D + 1)]))
                dz = (plsc.load_gather(poscb, [cl8 + 2])
                      - plsc.load_gather(ts, [gi, zvec + (D + 2)]))
                sh = _sh_vectors(dx, dy, dz)

                def i_body(i, _):
                    sv = plsc.load_gather(ts, [gi, zvec + i])
                    plsc.addupdate_scatter(acc, [addr + i], sv)
                    a2 = addr + (D + 16 * i)
                    for j in range(16):
                        plsc.addupdate_scatter(acc, [a2 + j], sv * sh[j])
                    return 0
                lax.fori_loop(0, D + dz2, i_body, 0)
                return 0
            lax.fori_loop(0, nbs, sbody, 0)
            pltpu.sync_copy(acc.at[pl.ds(0, ACCSZ)],
                            aggr_hbm.at[pl.ds(chunk * ACCSZ, ACCSZ)])
        return 0
    lax.fori_loop(0, PASSES + dz2, pass_body, 0)


def _sc_scan(row, col):
    f = functools.partial(
        pl.kernel,
        mesh=_mesh(),
        compiler_params=pltpu.CompilerParams(needs_layout_passes=False),
        out_type=(
            jax.ShapeDtypeStruct((NCHUNK * QCAP,), jnp.int32),
            jax.ShapeDtypeStruct((NCHUNK * 16,), jnp.int32),
        ),
        scratch_types=[
            pltpu.VMEM((2 * SBLK,), jnp.int32),
            pltpu.VMEM((2 * SBLK,), jnp.int32),
            pltpu.VMEM((QCAP,), jnp.int32),
            pltpu.VMEM((QCAP,), jnp.int32),
            pltpu.VMEM((16,), jnp.int32),
            pltpu.VMEM((16,), jnp.int32),
            pltpu.SemaphoreType.DMA,
            pltpu.SemaphoreType.DMA,
        ],
    )(_scan_body)
    return f(row, col)


def _sc_aggregate(tbl, pos8, qpk, cnt):
    f = functools.partial(
        pl.kernel,
        mesh=_mesh(),
        compiler_params=pltpu.CompilerParams(needs_layout_passes=False),
        out_type=jax.ShapeDtypeStruct((NPAD * ROW,), jnp.float32),
        scratch_types=[
            pltpu.VMEM((ACCSZ + 352,), jnp.float32),
            pltpu.VMEM((QCAP,), jnp.int32),
            pltpu.VMEM((2 * BATCH,), jnp.int32),
            pltpu.VMEM((2 * BATCH, 128), jnp.float32),
            pltpu.VMEM((NPC * 8,), jnp.float32),
            pltpu.VMEM((16,), jnp.int32),
            pltpu.SemaphoreType.DMA,
            pltpu.SemaphoreType.DMA,
            pltpu.SemaphoreType.DMA,
        ],
    )(_aggr_body)
    return f(tbl, pos8, qpk, cnt)


def _mlp_body(x_ref, wpre_ref, bpre_ref, wpost_ref, bpost_ref, wsc_ref,
              bsc_ref, o_ref):
    x = x_ref[...]
    h = jnp.maximum(
        jnp.dot(x, wpre_ref[...], preferred_element_type=jnp.float32)
        + bpre_ref[...][None, :], 0.0)
    h = jnp.dot(h, wpost_ref[...], preferred_element_type=jnp.float32) \
        + bpost_ref[...][None, :]
    sc = jnp.dot(x, wsc_ref[...], preferred_element_type=jnp.float32) \
        + bsc_ref[...][None, :]
    o_ref[...] = sc + h


def _mlp(x, W_pre, b_pre, W_post, b_post, W_sc, b_sc):
    n, d = x.shape
    blk = 2000
    return pl.pallas_call(
        _mlp_body,
        grid=(n // blk,),
        in_specs=[
            pl.BlockSpec((blk, d), lambda i: (i, 0)),
            pl.BlockSpec((d, d), lambda i: (0, 0)),
            pl.BlockSpec((d,), lambda i: (0,)),
            pl.BlockSpec((d, d), lambda i: (0, 0)),
            pl.BlockSpec((d,), lambda i: (0,)),
            pl.BlockSpec((d, d), lambda i: (0, 0)),
            pl.BlockSpec((d,), lambda i: (0,)),
        ],
        out_specs=pl.BlockSpec((blk, d), lambda i: (i, 0)),
        out_shape=jax.ShapeDtypeStruct((n, d), jnp.float32),
    )(x, W_pre, b_pre, W_post, b_post, W_sc, b_sc)


def kernel(x, edge_index, positions, W_pre, b_pre, W_post, b_post, W_sc,
           b_sc):
    row = edge_index[0]
    col = edge_index[1]
    qpk, cnt = _sc_scan(row, col)
    tbl = jnp.pad(jnp.concatenate([x, positions], axis=1),
                  ((0, 0), (0, 128 - D - 3)))
    pos8 = jnp.pad(positions, ((0, NPAD - N), (0, 5))).reshape(-1)
    aggr = _sc_aggregate(tbl, pos8, qpk, cnt)
    aggr = aggr.reshape(NPAD, ROW)[:N]
    out = _mlp(x, W_pre, b_pre, W_post, b_post, W_sc, b_sc)
    return out, aggr


# speedup vs baseline: 1.9350x; 1.1091x over previous
import functools

import jax
import jax.numpy as jnp
from jax import lax
from jax.experimental import pallas as pl
from jax.experimental.pallas import tpu as pltpu
from jax.experimental.pallas import tpu_sc as plsc

N = 50000
E = 800000
D = 20
NSH = 16
ROW = D + D * NSH
NPC = 320
NCHUNK = (N + NPC - 1) // NPC
NPAD = NCHUNK * NPC
NWORK = 32
PASSES = (NCHUNK + NWORK - 1) // NWORK
SBLK = 4000
NBLK = E // SBLK
QCAP = 8192
LANECAP = 512
BATCH = 16
ACCSZ = NPC * ROW


def _sh_vectors(dx, dy, dz):
    n2 = dx * dx + dy * dy + dz * dz
    i = plsc.bitcast(n2, jnp.int32)
    i = jnp.int32(0x5F3759DF) - (i >> 1)
    r = plsc.bitcast(i, jnp.float32)
    for _ in range(3):
        r = r * (1.5 - 0.5 * n2 * r * r)
    nrm = n2 * r
    s = jnp.maximum(nrm, 1e-12)
    x = dx / s
    y = dy / s
    z = dz / s
    x2 = x * x
    y2 = y * y
    z2 = z * z
    return [
        jnp.full((16,), 0.28209479177387814, jnp.float32),
        0.4886025119029199 * y,
        0.4886025119029199 * z,
        0.4886025119029199 * x,
        1.0925484305920792 * x * y,
        1.0925484305920792 * y * z,
        0.31539156525252005 * (3.0 * z2 - 1.0),
        1.0925484305920792 * x * z,
        0.5462742152960396 * (x2 - y2),
        0.5900435899266435 * y * (3.0 * x2 - y2),
        2.890611442640554 * x * y * z,
        0.4570457994644658 * y * (5.0 * z2 - 1.0),
        0.3731763325901154 * z * (5.0 * z2 - 3.0),
        0.4570457994644658 * x * (5.0 * z2 - 1.0),
        1.445305721320277 * z * (x2 - y2),
        0.5900435899266435 * x * (3.0 * x2 - y2),
    ]


def _mesh():
    return plsc.VectorSubcoreMesh(core_axis_name="c", subcore_axis_name="s")


def _scan_body(row_hbm, col_hbm, qpk_hbm, cnt_hbm, colbuf, rowbuf,
               qr, scb, cbuf, csem, rsem):
    cid = lax.axis_index("c")
    sid = lax.axis_index("s")
    wid = sid * 2 + cid
    iota = lax.broadcasted_iota(jnp.int32, (16,), 0)
    zvec = jnp.zeros((16,), jnp.int32)

    def issue_scan_st(b, off):
        pltpu.async_copy(col_hbm.at[pl.ds(b * SBLK, SBLK)],
                         colbuf.at[pl.ds(off * SBLK, SBLK)], csem)
        pltpu.async_copy(row_hbm.at[pl.ds(b * SBLK, SBLK)],
                         rowbuf.at[pl.ds(off * SBLK, SBLK)], rsem)

    def issue_scan(b, par):
        @pl.when(par == 0)
        def _():
            issue_scan_st(b, 0)

        @pl.when(par != 0)
        def _():
            issue_scan_st(b, 1)

    def wait_scan_st(off):
        pltpu.make_async_copy(col_hbm.at[pl.ds(0, SBLK)],
                              colbuf.at[pl.ds(off * SBLK, SBLK)], csem).wait()
        pltpu.make_async_copy(row_hbm.at[pl.ds(0, SBLK)],
                              rowbuf.at[pl.ds(off * SBLK, SBLK)], rsem).wait()

    def wait_scan(par):
        @pl.when(par == 0)
        def _():
            wait_scan_st(0)

        @pl.when(par != 0)
        def _():
            wait_scan_st(1)

    def qz(zi, _):
        qr[pl.ds(zi * 16, 16)] = zvec
        return 0
    lax.fori_loop(0, (5 * QCAP + 16) // 16, qz, 0)
    dz = qr[pl.ds(0, 16)][0]

    issue_scan_st(0, 0)

    def blk(b, cs):
        par = lax.rem(b, 2)

        @pl.when(b + 1 < NBLK)
        def _():
            issue_scan(b + 1, lax.rem(b + 1, 2))
        wait_scan(par)
        pbase = par * SBLK

        def vec(v, cs):
            c0, c1, c2, c3, c4 = cs
            cv = colbuf[pl.ds(pbase + v * 16, 16)]
            rv = rowbuf[pl.ds(pbase + v * 16, 16)]
            ch = ((cv >> 6) * 52429) >> 18
            hit = (ch & 31) == wid
            p = ch >> 5
            packed = ((cv - ch * NPC) << 16) | rv
            csel = jnp.where(p == 0, c0,
                             jnp.where(p == 1, c1,
                                       jnp.where(p == 2, c2,
                                                 jnp.where(p == 3, c3, c4))))
            offs = jnp.where(hit, p * QCAP + csel * 16 + iota,
                             5 * QCAP + iota)
            plsc.store_scatter(qr, [offs], packed)
            cap = jnp.int32(LANECAP - 1)
            c0 = jnp.minimum(c0 + (hit & (p == 0)).astype(jnp.int32), cap)
            c1 = jnp.minimum(c1 + (hit & (p == 1)).astype(jnp.int32), cap)
            c2 = jnp.minimum(c2 + (hit & (p == 2)).astype(jnp.int32), cap)
            c3 = jnp.minimum(c3 + (hit & (p == 3)).astype(jnp.int32), cap)
            c4 = jnp.minimum(c4 + (hit & (p == 4)).astype(jnp.int32), cap)
            return (c0, c1, c2, c3, c4)
        return lax.fori_loop(0, SBLK // 16 + dz, vec, cs)

    cs = lax.fori_loop(0, NBLK + dz, blk, (zvec, zvec, zvec, zvec, zvec))
    for p2 in range(PASSES):
        chunk = p2 * NWORK + wid

        @pl.when(chunk < NCHUNK)
        def _wout(p2=p2, chunk=chunk):
            pltpu.sync_copy(qr.at[pl.ds(p2 * QCAP, QCAP)],
                            qpk_hbm.at[pl.ds(chunk * QCAP, QCAP)])
            cbuf[pl.ds(0, 16)] = cs[p2]
            pltpu.sync_copy(cbuf, cnt_hbm.at[pl.ds(chunk * 16, 16)])


def _aggr_body(tbl_hbm, pos_hbm, qpk_hbm, cnt_hbm, aggr_hbm, acc,
               qp, idxr, ts, poscb, cbuf, qsem, xsem, pcsem):
    cid = lax.axis_index("c")
    sid = lax.axis_index("s")
    wid = sid * 2 + cid
    iota = lax.broadcasted_iota(jnp.int32, (16,), 0)
    zvec = jnp.zeros((16,), jnp.int32)
    acc[pl.ds(0, 16)] = jnp.zeros((16,), jnp.float32)
    dz2 = plsc.bitcast(acc[pl.ds(0, 16)], jnp.int32)[0]

    def pass_body(p, _):
        chunk = p * NWORK + wid
        lo = chunk * NPC

        @pl.when(chunk < NCHUNK)
        def _run():
            pltpu.sync_copy(cnt_hbm.at[pl.ds(chunk * 16, 16)], cbuf)
            pltpu.sync_copy(pos_hbm.at[pl.ds(lo * 8, NPC * 8)], poscb)
            cntv = cbuf[pl.ds(0, 16)]
            h1 = pltpu.async_copy(qpk_hbm.at[pl.ds(chunk * QCAP, QCAP)],
                                  qp, qsem)

            def zbody(zi, _):
                acc[pl.ds(zi * 16, 16)] = jnp.zeros((16,), jnp.float32)
                return 0
            lax.fori_loop(0, (ACCSZ + 352) // 16 + dz2, zbody, 0)
            h1.wait()

            nbs = cntv[0]
            for k in range(1, 16):
                nbs = jnp.maximum(nbs, cntv[k])

            def issue_slot(b, off):
                pv = qp[pl.ds(b * 16, 16)]
                idxr[pl.ds(off * 16, 16)] = pv & jnp.int32(0xFFFF)
                pltpu.async_copy(
                    tbl_hbm.at[idxr.at[pl.ds(off * 16, 16)]],
                    ts.at[pl.ds(off * 16, 16)], xsem)

            def issue_dyn(b, par):
                @pl.when(par == 0)
                def _():
                    issue_slot(b, 0)

                @pl.when(par != 0)
                def _():
                    issue_slot(b, 1)

            def wait_slot(off):
                pltpu.make_async_copy(
                    tbl_hbm.at[idxr.at[pl.ds(off * 16, 16)]],
                    ts.at[pl.ds(off * 16, 16)], xsem).wait()

            def wait_dyn(par):
                @pl.when(par == 0)
                def _():
                    wait_slot(0)

                @pl.when(par != 0)
                def _():
                    wait_slot(1)

            @pl.when(nbs > 0)
            def _():
                issue_slot(0, 0)

            def sbody(b, _):
                par = lax.rem(b, 2)

                @pl.when(b + 1 < nbs)
                def _():
                    issue_dyn(b + 1, lax.rem(b + 1, 2))
                wait_dyn(par)
                msk = cntv > b
                pv = qp[pl.ds(b * 16, 16)]
                cl = pv >> 16
                addr = jnp.where(msk, cl * ROW, jnp.int32(ACCSZ))
                gi = par * 16 + iota
                cl8 = cl * 8
                dx = (plsc.load_gather(poscb, [cl8])
                      - plsc.load_gather(ts, [gi, zvec + D]))
                dy = (plsc.load_gather(poscb, [cl8 + 1])
                      - plsc.load_gather(ts, [gi, zvec + (D + 1)]))
                dz = (plsc.load_gather(poscb, [cl8 + 2])
                      - plsc.load_gather(ts, [gi, zvec + (D + 2)]))
                sh = _sh_vectors(dx, dy, dz)

                def i_body(i, _):
                    sv = plsc.load_gather(ts, [gi, zvec + i])
                    plsc.addupdate_scatter(acc, [addr + i], sv)
                    a2 = addr + (D + 16 * i)
                    for j in range(16):
                        plsc.addupdate_scatter(acc, [a2 + j], sv * sh[j])
                    return 0
                lax.fori_loop(0, D + dz2, i_body, 0)
                return 0
            lax.fori_loop(0, nbs, sbody, 0)
            pltpu.sync_copy(acc.at[pl.ds(0, ACCSZ)],
                            aggr_hbm.at[pl.ds(chunk * ACCSZ, ACCSZ)])
        return 0
    lax.fori_loop(0, PASSES + dz2, pass_body, 0)


def _sc_scan(row, col):
    f = functools.partial(
        pl.kernel,
        mesh=_mesh(),
        compiler_params=pltpu.CompilerParams(needs_layout_passes=False),
        out_type=(
            jax.ShapeDtypeStruct((NCHUNK * QCAP,), jnp.int32),
            jax.ShapeDtypeStruct((NCHUNK * 16,), jnp.int32),
        ),
        scratch_types=[
            pltpu.VMEM((2 * SBLK,), jnp.int32),
            pltpu.VMEM((2 * SBLK,), jnp.int32),
            pltpu.VMEM((5 * QCAP + 16,), jnp.int32),
            pltpu.VMEM((16,), jnp.int32),
            pltpu.VMEM((16,), jnp.int32),
            pltpu.SemaphoreType.DMA,
            pltpu.SemaphoreType.DMA,
        ],
    )(_scan_body)
    return f(row, col)


def _sc_aggregate(tbl, pos8, qpk, cnt):
    f = functools.partial(
        pl.kernel,
        mesh=_mesh(),
        compiler_params=pltpu.CompilerParams(needs_layout_passes=False),
        out_type=jax.ShapeDtypeStruct((NPAD * ROW,), jnp.float32),
        scratch_types=[
            pltpu.VMEM((ACCSZ + 352,), jnp.float32),
            pltpu.VMEM((QCAP,), jnp.int32),
            pltpu.VMEM((2 * BATCH,), jnp.int32),
            pltpu.VMEM((2 * BATCH, 128), jnp.float32),
            pltpu.VMEM((NPC * 8,), jnp.float32),
            pltpu.VMEM((16,), jnp.int32),
            pltpu.SemaphoreType.DMA,
            pltpu.SemaphoreType.DMA,
            pltpu.SemaphoreType.DMA,
        ],
    )(_aggr_body)
    return f(tbl, pos8, qpk, cnt)


def _mlp_body(x_ref, wpre_ref, bpre_ref, wpost_ref, bpost_ref, wsc_ref,
              bsc_ref, o_ref):
    x = x_ref[...]
    h = jnp.maximum(
        jnp.dot(x, wpre_ref[...], preferred_element_type=jnp.float32)
        + bpre_ref[...][None, :], 0.0)
    h = jnp.dot(h, wpost_ref[...], preferred_element_type=jnp.float32) \
        + bpost_ref[...][None, :]
    sc = jnp.dot(x, wsc_ref[...], preferred_element_type=jnp.float32) \
        + bsc_ref[...][None, :]
    o_ref[...] = sc + h


def _mlp(x, W_pre, b_pre, W_post, b_post, W_sc, b_sc):
    n, d = x.shape
    blk = 2000
    return pl.pallas_call(
        _mlp_body,
        grid=(n // blk,),
        in_specs=[
            pl.BlockSpec((blk, d), lambda i: (i, 0)),
            pl.BlockSpec((d, d), lambda i: (0, 0)),
            pl.BlockSpec((d,), lambda i: (0,)),
            pl.BlockSpec((d, d), lambda i: (0, 0)),
            pl.BlockSpec((d,), lambda i: (0,)),
            pl.BlockSpec((d, d), lambda i: (0, 0)),
            pl.BlockSpec((d,), lambda i: (0,)),
        ],
        out_specs=pl.BlockSpec((blk, d), lambda i: (i, 0)),
        out_shape=jax.ShapeDtypeStruct((n, d), jnp.float32),
    )(x, W_pre, b_pre, W_post, b_post, W_sc, b_sc)


def kernel(x, edge_index, positions, W_pre, b_pre, W_post, b_post, W_sc,
           b_sc):
    row = edge_index[0]
    col = edge_index[1]
    qpk, cnt = _sc_scan(row, col)
    tbl = jnp.pad(jnp.concatenate([x, positions], axis=1),
                  ((0, 0), (0, 128 - D - 3)))
    pos8 = jnp.pad(positions, ((0, NPAD - N), (0, 5))).reshape(-1)
    aggr = _sc_aggregate(tbl, pos8, qpk, cnt)
    aggr = aggr.reshape(NPAD, ROW)[:N]
    out = _mlp(x, W_pre, b_pre, W_post, b_post, W_sc, b_sc)
    return out, aggr
